# baseline (speedup 1.0000x reference)
import functools

import numpy as np
import jax
import jax.numpy as jnp
from jax import lax
from jax.experimental import pallas as pl
from jax.experimental.pallas import tpu as pltpu
from jax.experimental.pallas import tpu_sc as plsc

_SEQ = 8192
_D = 512
_LANES = 16
_NC = 2
_NS = 16
_NW = _NC * _NS
_BPW = _SEQ // _NW
_C = 32
_NCH = _BPW // _C


def _positional_encodings_np(seq_len, d):
    pos = np.arange(seq_len, dtype=np.float64)[:, None]
    hid = np.arange(d, dtype=np.float64)[None, :]
    angles = pos / np.power(10000.0, 2.0 * (np.floor(hid / 2.0)) / d)
    pe = np.array(angles)
    pe[:, 0::2] = np.sin(angles[:, 0::2])
    pe[:, 1::2] = np.cos(angles[:, 1::2])
    return pe.astype(np.float32)


_PE_NP = _positional_encodings_np(_SEQ, _D)


def _body(x_hbm, table_hbm, pe_hbm, out_hbm, idx_v, rows_v, pe_v, sem):
    wid = lax.axis_index("s") * _NC + lax.axis_index("c")
    base = wid * _BPW
    for ch in range(_NCH):
        pltpu.sync_copy(x_hbm.at[pl.ds(base + ch * _C, _C)], idx_v.at[ch])
    for ch in range(_NCH):
        rbase = base + ch * _C
        gather = pltpu.async_copy(table_hbm.at[idx_v.at[ch]], rows_v, sem)
        pltpu.sync_copy(pe_hbm.at[pl.ds(rbase, _C)], pe_v)
        gather.wait()

        def row_body(r, _):
            def col_body(j, _):
                off = j * _LANES
                rows_v[r, pl.ds(off, _LANES)] = (
                    rows_v[r, pl.ds(off, _LANES)] + pe_v[r, pl.ds(off, _LANES)]
                )
                return 0

            lax.fori_loop(0, _D // _LANES, col_body, 0)
            return 0

        lax.fori_loop(0, _C, row_body, 0)
        pltpu.sync_copy(rows_v, out_hbm.at[pl.ds(rbase, _C)])


_sc_kernel = functools.partial(
    pl.kernel,
    out_type=jax.ShapeDtypeStruct((_SEQ, _D), jnp.float32),
    mesh=plsc.VectorSubcoreMesh(core_axis_name="c", subcore_axis_name="s"),
    scratch_types=[
        pltpu.VMEM((_NCH, _C), jnp.int32),
        pltpu.VMEM((_C, _D), jnp.float32),
        pltpu.VMEM((_C, _D), jnp.float32),
        pltpu.SemaphoreType.DMA,
    ],
)(_body)


def kernel(x, table):
    pe = jnp.asarray(_PE_NP)
    return _sc_kernel(x.astype(jnp.int32), table, pe)


# speedup vs baseline: 1.5932x; 1.5932x over previous
import functools

import numpy as np
import jax
import jax.numpy as jnp
from jax import lax
from jax.experimental import pallas as pl
from jax.experimental.pallas import tpu as pltpu
from jax.experimental.pallas import tpu_sc as plsc

_SEQ = 8192
_D = 512
_LANES = 16
_NC = 2
_NS = 16
_NW = _NC * _NS
_BPW = _SEQ // _NW
_C = 32
_NCH = _BPW // _C
_NBUF = 4
_PBUF = 2


def _positional_encodings_np(seq_len, d):
    pos = np.arange(seq_len, dtype=np.float64)[:, None]
    hid = np.arange(d, dtype=np.float64)[None, :]
    angles = pos / np.power(10000.0, 2.0 * (np.floor(hid / 2.0)) / d)
    pe = np.array(angles)
    pe[:, 0::2] = np.sin(angles[:, 0::2])
    pe[:, 1::2] = np.cos(angles[:, 1::2])
    return pe.astype(np.float32)


_PE_NP = _positional_encodings_np(_SEQ, _D)


def _body(x3_hbm, table_hbm, pe_hbm, out_hbm, idx_v, rows_v, pe_v,
          sem_g, sem_p, sem_o):
    wid = lax.axis_index("s") * _NC + lax.axis_index("c")
    base = wid * _BPW
    pltpu.sync_copy(x3_hbm.at[wid], idx_v)

    def start_gather(ch):
        return pltpu.async_copy(
            table_hbm.at[idx_v.at[ch]], rows_v.at[ch % _NBUF],
            sem_g.at[ch % _NBUF])

    def start_pe(ch):
        return pltpu.async_copy(
            pe_hbm.at[pl.ds(base + ch * _C, _C)], pe_v.at[ch % _PBUF],
            sem_p.at[ch % _PBUF])

    g, p, o = {}, {}, {}
    for ch in range(_PBUF):
        g[ch] = start_gather(ch)
        p[ch] = start_pe(ch)

    for ch in range(_NCH):
        b = ch % _NBUF
        pb = ch % _PBUF
        nxt = ch + _PBUF
        if nxt < _NCH:
            if nxt - _NBUF >= 0:
                o[nxt - _NBUF].wait()
            g[nxt] = start_gather(nxt)
        g[ch].wait()
        p[ch].wait()

        @plsc.parallel_loop(0, _C, step=1, unroll=2)
        def _add(r):
            for j in range(_D // _LANES):
                off = j * _LANES
                rows_v[b, r, pl.ds(off, _LANES)] = (
                    rows_v[b, r, pl.ds(off, _LANES)]
                    + pe_v[pb, r, pl.ds(off, _LANES)]
                )

        o[ch] = pltpu.async_copy(
            rows_v.at[b], out_hbm.at[pl.ds(base + ch * _C, _C)], sem_o.at[b])
        if nxt < _NCH:
            p[nxt] = start_pe(nxt)

    for ch in range(_NCH - min(_NBUF, _NCH), _NCH):
        o[ch].wait()


_sc_kernel = functools.partial(
    pl.kernel,
    out_type=jax.ShapeDtypeStruct((_SEQ, _D), jnp.float32),
    mesh=plsc.VectorSubcoreMesh(core_axis_name="c", subcore_axis_name="s"),
    scratch_types=[
        pltpu.VMEM((_NCH, _C), jnp.int32),
        pltpu.VMEM((_NBUF, _C, _D), jnp.float32),
        pltpu.VMEM((_PBUF, _C, _D), jnp.float32),
        pltpu.SemaphoreType.DMA((_NBUF,)),
        pltpu.SemaphoreType.DMA((_PBUF,)),
        pltpu.SemaphoreType.DMA((_NBUF,)),
    ],
)(_body)


def kernel(x, table):
    pe = jnp.asarray(_PE_NP)
    x3 = x.astype(jnp.int32).reshape(_NW, _NCH, _C)
    return _sc_kernel(x3, table, pe)
